# baseline (speedup 1.0000x reference)
import functools

import jax
import jax.numpy as jnp
from jax import lax
from jax.experimental import pallas as pl
from jax.experimental.pallas import tpu as pltpu
from jax.experimental.pallas import tpu_sc as plsc

B, P, N, C_IN, C_OUT, W, H = 2, 12000, 100, 7, 64, 504, 504
EPS = 1e-5
BP = B * P
BPP = 24576
CELLS = W * H
TOT = B * CELLS

NB = 2
NCH = N // NB
LN = P


def _tc_body(x_ref, w_ref, a_ref, d_ref, feat_ref, acc_ref):
    nc = pl.program_id(0)
    m = None
    for nb in range(NB):
        xb = x_ref[nb].reshape(2 * C_IN, LN)
        z = lax.dot_general(
            w_ref[...], xb, (((1,), (0,)), ((), ())),
            preferred_element_type=jnp.float32)
        m = z if m is None else jnp.maximum(m, z)

    @pl.when(nc == 0)
    def _():
        acc_ref[...] = m

    @pl.when(nc > 0)
    def _():
        acc_ref[...] = jnp.maximum(acc_ref[...], m)

    @pl.when(nc == NCH - 1)
    def _():
        feat_ref[...] = jnp.maximum(
            acc_ref[...] * a_ref[...] + d_ref[...], 0.0)


def _tc_stage(x_t, w2, a2, d2):
    return pl.pallas_call(
        _tc_body,
        grid=(NCH,),
        in_specs=[
            pl.BlockSpec((NB, C_IN, 2, LN), lambda nc: (nc, 0, 0, 0)),
            pl.BlockSpec((128, 2 * C_IN), lambda nc: (0, 0)),
            pl.BlockSpec((128, 1), lambda nc: (0, 0)),
            pl.BlockSpec((128, 1), lambda nc: (0, 0)),
        ],
        out_specs=pl.BlockSpec((128, LN), lambda nc: (0, 0)),
        out_shape=jax.ShapeDtypeStruct((128, LN), jnp.float32),
        scratch_shapes=[pltpu.VMEM((128, LN), jnp.float32)],
    )(x_t, w2, a2, d2)


def _cells_body(idx_ref, cell_ref):
    i0 = idx_ref[0:1, :]
    ix = idx_ref[1:2, :]
    iy = idx_ref[2:3, :]
    valid = (i0 != 0) | (ix != 0) | (iy != 0)
    b = lax.broadcasted_iota(jnp.int32, (1, BPP), 1) // P
    cell = jnp.where(valid, b * CELLS + ix * H + iy, -1)
    cell_ref[...] = cell.reshape(1, 1, BPP)


def _cells_stage(idx_t):
    return pl.pallas_call(
        _cells_body,
        out_shape=jax.ShapeDtypeStruct((1, 1, BPP), jnp.int32),
    )(idx_t)


NW = 32
RANGE0 = 15872
SEENN = 16000
NBLK = BPP // 16
CHUNK = 128
WBUF = 16256


def _sc_dedup_body(cells_hbm, wpo_hbm, wco_hbm, cnto_hbm,
                   cells_v, seen_v, wp_v, wc_v, scr16_v):
    info = plsc.get_sparse_core_info()
    nc = info.num_cores
    wid = lax.axis_index("s") * nc + lax.axis_index("c")
    base = wid * RANGE0
    rng = RANGE0 + jnp.where(wid == NW - 1, TOT - NW * RANGE0, 0)

    pltpu.sync_copy(cells_hbm, cells_v)

    def sb(t, _):
        seen_v[pl.ds(t * 16, 16)] = jnp.zeros((16,), jnp.int32)
        return 0
    lax.fori_loop(0, SEENN // 16, sb, 0)

    lane = lax.broadcasted_iota(jnp.int32, (16,), 0)
    ones16 = jnp.ones((16,), jnp.int32)

    def dedup(t, cnt):
        i = (NBLK - 1) - t
        cvec = cells_v[pl.ds(i * 16, 16)]
        local = cvec - base
        inm = (local >= 0) & (local < rng)
        anyin = plsc.all_reduce_population_count(inm)[0]

        def heavy(c):
            key = jnp.where(inm, local * 16 + lane, 0x7FFE0000 + lane)
            sk, sv = plsc.sort_key_val(key, lane)
            nxt = sk[jnp.minimum(lane + 1, 15)]
            islast = ((sk >> 4) != (nxt >> 4)) | (lane == 15)
            plsc.store_scatter(scr16_v, [sv], islast.astype(jnp.int32))
            won1 = inm & (scr16_v[...] != 0)
            localc = jnp.clip(local, 0, SEENN - 1)
            seen = plsc.load_gather(seen_v, [localc], mask=won1)
            won = won1 & (seen == 0)
            plsc.store_scatter(seen_v, [localc], ones16, mask=won)
            pvec = i * 16 + lane
            plsc.store_compressed(wp_v.at[pl.ds(c, 16)], pvec, mask=won)
            plsc.store_compressed(wc_v.at[pl.ds(c, 16)], cvec, mask=won)
            nw = plsc.all_reduce_population_count(won)[0]
            return c + nw

        return lax.cond(anyin > 0, heavy, lambda c: c, cnt)

    cnt = lax.fori_loop(0, NBLK, dedup, 0)

    @pl.when(cnt > 0)
    def _():
        p0 = wp_v[pl.ds(0, 16)][0]
        c0 = wc_v[pl.ds(0, 16)][0]
        t0 = cnt // 16

        def pad(k, _):
            off = (t0 + k) * 16
            m = (off + lane) >= cnt
            wp_v[pl.ds(off, 16)] = jnp.where(m, p0, wp_v[pl.ds(off, 16)])
            wc_v[pl.ds(off, 16)] = jnp.where(m, c0, wc_v[pl.ds(off, 16)])
            return 0
        lax.fori_loop(0, 9, pad, 0)

    scr16_v[...] = jnp.full((16,), cnt, jnp.int32)
    pltpu.sync_copy(wp_v, wpo_hbm.at[wid])
    pltpu.sync_copy(wc_v, wco_hbm.at[wid])
    pltpu.sync_copy(scr16_v, cnto_hbm.at[wid])


def _sc_scatter_body(feat_hbm, wpo_hbm, wco_hbm, cnto_hbm, img_hbm,
                     wp_v, wc2_v, data_v, scr16_v, gsem, ssem):
    info = plsc.get_sparse_core_info()
    nc = info.num_cores
    wid = lax.axis_index("s") * nc + lax.axis_index("c")

    pltpu.sync_copy(cnto_hbm.at[wid], scr16_v)
    cnt = scr16_v[...][0]

    @pl.when(cnt > 0)
    def _():
        nch = (cnt + (CHUNK - 1)) // CHUNK
        pltpu.sync_copy(wpo_hbm.at[wid], wp_v)

        def fetch(j, _):
            pltpu.sync_copy(
                wco_hbm.at[wid].at[pl.ds(j * CHUNK, CHUNK)], wc2_v.at[j])
            return 0
        lax.fori_loop(0, nch, fetch, 0)

        def chunk(j, _):
            pltpu.async_copy(
                feat_hbm.at[wp_v.at[pl.ds(j * CHUNK, CHUNK)]], data_v,
                gsem).wait()
            pltpu.async_copy(
                data_v, img_hbm.at[wc2_v.at[j]], ssem).wait()
            return 0
        lax.fori_loop(0, nch, chunk, 0)


def _sc_dedup_stage(cells):
    mesh = plsc.VectorSubcoreMesh(core_axis_name="c", subcore_axis_name="s")
    return pl.kernel(
        _sc_dedup_body,
        out_type=(
            jax.ShapeDtypeStruct((NW, WBUF), jnp.int32),
            jax.ShapeDtypeStruct((NW, WBUF), jnp.int32),
            jax.ShapeDtypeStruct((NW, 16), jnp.int32),
        ),
        mesh=mesh,
        compiler_params=pltpu.CompilerParams(
            needs_layout_passes=False, use_tc_tiling_on_sc=True),
        scratch_types=[
            pltpu.VMEM((BPP,), jnp.int32),
            pltpu.VMEM((SEENN,), jnp.int32),
            pltpu.VMEM((WBUF,), jnp.int32),
            pltpu.VMEM((WBUF,), jnp.int32),
            pltpu.VMEM((16,), jnp.int32),
        ],
    )(cells)


def _sc_scatter_stage(feat, wpo, wco, cnto, img_ref):
    mesh = plsc.VectorSubcoreMesh(core_axis_name="c", subcore_axis_name="s")
    return pl.kernel(
        _sc_scatter_body,
        out_type=(),
        mesh=mesh,
        compiler_params=pltpu.CompilerParams(
            needs_layout_passes=False, use_tc_tiling_on_sc=True),
        scratch_types=[
            pltpu.VMEM((WBUF,), jnp.int32),
            pltpu.VMEM((WBUF // CHUNK, CHUNK), jnp.int32),
            pltpu.VMEM((CHUNK, 128), jnp.float32),
            pltpu.VMEM((16,), jnp.int32),
            pltpu.SemaphoreType.DMA,
            pltpu.SemaphoreType.DMA,
        ],
    )(feat, wpo, wco, cnto, img_ref)


def kernel(pillar_points, pillar_indices, conv_w, conv_b, bn_gamma, bn_beta,
           bn_mean, bn_var):
    a = bn_gamma * lax.rsqrt(bn_var + EPS)
    s = jnp.where(a >= 0, 1.0, -1.0)
    w2s = conv_w * s[:, None]
    w2 = jnp.zeros((2 * C_OUT, 2 * C_IN), jnp.float32)
    for bb in range(2):
        w2 = w2.at[bb * C_OUT:(bb + 1) * C_OUT, bb::2].set(w2s)
    a2 = jnp.tile(a * s, 2)[:, None]
    d2 = jnp.tile(a * (conv_b - bn_mean) + bn_beta, 2)[:, None]

    x_t = pillar_points.transpose(2, 3, 0, 1)
    idx_t = jnp.pad(
        pillar_indices.reshape(BP, 3), ((0, BPP - BP), (0, 5))).T

    cells = _cells_stage(idx_t)
    wpo, wco, cnto = _sc_dedup_stage(cells.reshape(BPP))
    feat_t = _tc_stage(x_t, w2, a2, d2)
    feat = jnp.pad(
        feat_t.reshape(2, C_OUT, P).transpose(0, 2, 1).reshape(BP, C_OUT),
        ((0, 0), (0, 64)))
    img_ref = jax.new_ref(jnp.zeros((TOT, 128), jnp.float32))
    _sc_scatter_stage(feat, wpo, wco, cnto, img_ref)
    img = img_ref[...]
    return img.reshape(B, W, H, 2 * C_OUT)[:, :, :, :C_OUT]


# speedup vs baseline: 1.0184x; 1.0184x over previous
import jax
import jax.numpy as jnp
from jax import lax
from jax.experimental import pallas as pl
from jax.experimental.pallas import tpu as pltpu
from jax.experimental.pallas import tpu_sc as plsc

B, P, N, C_IN, C_OUT, W, H = 2, 12000, 100, 7, 64, 504, 504
EPS = 1e-5
BP = B * P
BPP = 24576
CELLS = W * H
TOT = B * CELLS

NB = 2
NCH = N // NB
LN = P


def _tc_body(x_ref, idx_ref, w_ref, a_ref, d_ref, feat_ref, cell_ref,
             acc_ref):
    nc = pl.program_id(0)
    m = None
    for nb in range(NB):
        xb = x_ref[nb].reshape(2 * C_IN, LN)
        z = lax.dot_general(
            w_ref[...], xb, (((1,), (0,)), ((), ())),
            preferred_element_type=jnp.float32)
        m = z if m is None else jnp.maximum(m, z)

    @pl.when(nc == 0)
    def _():
        acc_ref[...] = m

    @pl.when(nc > 0)
    def _():
        acc_ref[...] = jnp.maximum(acc_ref[...], m)

    @pl.when(nc == NCH - 1)
    def _():
        feat_ref[...] = jnp.maximum(
            acc_ref[...] * a_ref[...] + d_ref[...], 0.0)

    @pl.when(nc == 0)
    def _():
        i0 = idx_ref[0:1, :]
        ix = idx_ref[1:2, :]
        iy = idx_ref[2:3, :]
        valid = (i0 != 0) | (ix != 0) | (iy != 0)
        b = lax.broadcasted_iota(jnp.int32, (1, BPP), 1) // P
        cell = jnp.where(valid, b * CELLS + ix * H + iy, -1)
        cell_ref[...] = cell.reshape(1, 1, BPP)


def _tc_stage(x_t, idx_t, w2, a2, d2):
    return pl.pallas_call(
        _tc_body,
        grid=(NCH,),
        in_specs=[
            pl.BlockSpec((NB, C_IN, 2, LN), lambda nc: (nc, 0, 0, 0)),
            pl.BlockSpec((8, BPP), lambda nc: (0, 0)),
            pl.BlockSpec((128, 2 * C_IN), lambda nc: (0, 0)),
            pl.BlockSpec((128, 1), lambda nc: (0, 0)),
            pl.BlockSpec((128, 1), lambda nc: (0, 0)),
        ],
        out_specs=[
            pl.BlockSpec((128, LN), lambda nc: (0, 0)),
            pl.BlockSpec((1, 1, BPP), lambda nc: (0, 0, 0)),
        ],
        out_shape=[
            jax.ShapeDtypeStruct((128, LN), jnp.float32),
            jax.ShapeDtypeStruct((1, 1, BPP), jnp.int32),
        ],
        scratch_shapes=[pltpu.VMEM((128, LN), jnp.float32)],
    )(x_t, idx_t, w2, a2, d2)


NW = 32
RANGE0 = 15872
SEENN = 16000
NBLK = BPP // 16
CHUNK = 128
WBUF = 16256


def _sc_body(feat_hbm, cells_hbm, img_hbm,
             cells_v, seen_v, wp_v, wc_v, wc2_v, data_v, scr16_v,
             gsem, ssem):
    info = plsc.get_sparse_core_info()
    nc = info.num_cores
    wid = lax.axis_index("s") * nc + lax.axis_index("c")
    base = wid * RANGE0
    rng = RANGE0 + jnp.where(wid == NW - 1, TOT - NW * RANGE0, 0)

    pltpu.sync_copy(cells_hbm, cells_v)

    def sb(t, _):
        seen_v[pl.ds(t * 16, 16)] = jnp.zeros((16,), jnp.int32)
        return 0
    lax.fori_loop(0, SEENN // 16, sb, 0)

    lane = lax.broadcasted_iota(jnp.int32, (16,), 0)
    ones16 = jnp.ones((16,), jnp.int32)

    def dedup(t, cnt):
        i = (NBLK - 1) - t
        cvec = cells_v[pl.ds(i * 16, 16)]
        local = cvec - base
        inm = (local >= 0) & (local < rng)
        anyin = plsc.all_reduce_population_count(inm)[0]

        def heavy(c):
            key = jnp.where(inm, local * 16 + lane, 0x7FFE0000 + lane)
            sk, sv = plsc.sort_key_val(key, lane)
            nxt = sk[jnp.minimum(lane + 1, 15)]
            islast = ((sk >> 4) != (nxt >> 4)) | (lane == 15)
            plsc.store_scatter(scr16_v, [sv], islast.astype(jnp.int32))
            won1 = inm & (scr16_v[...] != 0)
            localc = jnp.clip(local, 0, SEENN - 1)
            seen = plsc.load_gather(seen_v, [localc], mask=won1)
            won = won1 & (seen == 0)
            plsc.store_scatter(seen_v, [localc], ones16, mask=won)
            pvec = i * 16 + lane
            plsc.store_compressed(wp_v.at[pl.ds(c, 16)], pvec, mask=won)
            plsc.store_compressed(wc_v.at[pl.ds(c, 16)], cvec, mask=won)
            nw = plsc.all_reduce_population_count(won)[0]
            return c + nw

        return lax.cond(anyin > 0, heavy, lambda c: c, cnt)

    cnt = lax.fori_loop(0, NBLK, dedup, 0)

    @pl.when(cnt > 0)
    def _():
        p0 = wp_v[pl.ds(0, 16)][0]
        c0 = wc_v[pl.ds(0, 16)][0]
        t0 = cnt // 16

        def pad(k, _):
            off = (t0 + k) * 16
            m = (off + lane) >= cnt
            wp_v[pl.ds(off, 16)] = jnp.where(m, p0, wp_v[pl.ds(off, 16)])
            wc_v[pl.ds(off, 16)] = jnp.where(m, c0, wc_v[pl.ds(off, 16)])
            return 0
        lax.fori_loop(0, 9, pad, 0)

        nch = (cnt + (CHUNK - 1)) // CHUNK

        def repack(j, _):
            for k in range(CHUNK // 16):
                wc2_v[j, pl.ds(k * 16, 16)] = wc_v[
                    pl.ds(j * CHUNK + k * 16, 16)]
            return 0
        lax.fori_loop(0, nch, repack, 0)

        def chunk(j, _):
            pltpu.async_copy(
                feat_hbm.at[wp_v.at[pl.ds(j * CHUNK, CHUNK)]], data_v,
                gsem).wait()
            pltpu.async_copy(
                data_v, img_hbm.at[wc2_v.at[j]], ssem).wait()
            return 0
        lax.fori_loop(0, nch, chunk, 0)


def _sc_stage(feat, cells, img_ref):
    mesh = plsc.VectorSubcoreMesh(core_axis_name="c", subcore_axis_name="s")
    return pl.kernel(
        _sc_body,
        out_type=(),
        mesh=mesh,
        compiler_params=pltpu.CompilerParams(
            needs_layout_passes=False, use_tc_tiling_on_sc=True),
        scratch_types=[
            pltpu.VMEM((BPP,), jnp.int32),
            pltpu.VMEM((SEENN,), jnp.int32),
            pltpu.VMEM((WBUF,), jnp.int32),
            pltpu.VMEM((WBUF,), jnp.int32),
            pltpu.VMEM((WBUF // CHUNK, CHUNK), jnp.int32),
            pltpu.VMEM((CHUNK, 128), jnp.float32),
            pltpu.VMEM((16,), jnp.int32),
            pltpu.SemaphoreType.DMA,
            pltpu.SemaphoreType.DMA,
        ],
    )(feat, cells, img_ref)


def kernel(pillar_points, pillar_indices, conv_w, conv_b, bn_gamma, bn_beta,
           bn_mean, bn_var):
    a = bn_gamma * lax.rsqrt(bn_var + EPS)
    s = jnp.where(a >= 0, 1.0, -1.0)
    w2s = conv_w * s[:, None]
    w2 = jnp.zeros((2 * C_OUT, 2 * C_IN), jnp.float32)
    for bb in range(2):
        w2 = w2.at[bb * C_OUT:(bb + 1) * C_OUT, bb::2].set(w2s)
    a2 = jnp.tile(a * s, 2)[:, None]
    d2 = jnp.tile(a * (conv_b - bn_mean) + bn_beta, 2)[:, None]

    x_t = pillar_points.transpose(2, 3, 0, 1)
    idx_t = jnp.pad(
        pillar_indices.reshape(BP, 3), ((0, BPP - BP), (0, 5))).T

    feat_t, cells = _tc_stage(x_t, idx_t, w2, a2, d2)
    feat = jnp.pad(
        feat_t.reshape(2, C_OUT, P).transpose(0, 2, 1).reshape(BP, C_OUT),
        ((0, 0), (0, 64)))
    img_ref = jax.new_ref(jnp.zeros((TOT, 128), jnp.float32))
    _sc_stage(feat, cells.reshape(BPP), img_ref)
    img = img_ref[...]
    return img.reshape(B, W, H, 2 * C_OUT)[:, :, :, :C_OUT]
